# Initial kernel scaffold; baseline (speedup 1.0000x reference)
import functools

import jax
import jax.numpy as jnp
from jax import lax
from jax.experimental import pallas as pl
from jax.experimental.pallas import tpu as pltpu
from jax.experimental.pallas import tpu_sc as plsc

HEADS = 8
DK = 16
ACC_W = 144


def _qkv_body(x_ref, wq_ref, wk_ref, wv_ref, bd_ref, bq_ref, bk_ref, bv_ref,
              qu_ref, kv_ref):
    xb = x_ref[...]
    q = jnp.dot(xb, wq_ref[...], preferred_element_type=jnp.float32) + bq_ref[...]
    k = jnp.dot(xb, wk_ref[...], preferred_element_type=jnp.float32) + bk_ref[...]
    v = jnp.dot(xb, wv_ref[...], preferred_element_type=jnp.float32) + bv_ref[...]
    u = jnp.dot(q, bd_ref[...], preferred_element_type=jnp.float32)
    qu_ref[:, 0:128] = q
    qu_ref[:, 128:256] = u
    kv_ref[:, 0:128] = k
    kv_ref[:, 128:256] = v


def _gamma_body(ea_ref, wfg_ref, bfg_ref, g_ref):
    g_ref[...] = (jnp.dot(ea_ref[...], wfg_ref[...],
                          preferred_element_type=jnp.float32) + bfg_ref[...])


def _combine_body(parts_ref, x_ref, wo_ref, bo_ref, lng_ref, lnb_ref, o_ref):
    acc = parts_ref[0] + parts_ref[1]
    num = acc[:, 0:128]
    s = acc[:, 128:136]
    recip = 1.0 / (s + 1e-16)
    rep = jnp.concatenate(
        [jnp.broadcast_to(recip[:, h:h + 1], (num.shape[0], DK))
         for h in range(HEADS)], axis=1)
    att = num * rep
    y = jnp.dot(att, wo_ref[...], preferred_element_type=jnp.float32) + bo_ref[...]
    res = x_ref[...] + y
    mean = jnp.mean(res, axis=1, keepdims=True)
    cen = res - mean
    var = jnp.mean(cen * cen, axis=1, keepdims=True)
    o_ref[...] = lng_ref[...] * cen * lax.rsqrt(var + 1e-5) + lnb_ref[...]


def _make_edge_kernel(N, E, D):
    info = plsc.get_sparse_core_info()
    NC, NS = info.num_cores, info.num_subcores
    NW = NC * NS
    EP = E // NW
    C = 80
    assert E % NW == 0 and EP % C == 0 and N % NS == 0
    CH = EP // C
    NR = N // NS

    mesh = plsc.VectorSubcoreMesh(core_axis_name="c", subcore_axis_name="s")

    @functools.partial(
        pl.kernel,
        mesh=mesh,
        out_type=jax.ShapeDtypeStruct((NC, N, ACC_W), jnp.float32),
        scratch_types=[
            pltpu.VMEM_SHARED((N, ACC_W), jnp.float32),
            pltpu.VMEM((C,), jnp.int32),
            pltpu.VMEM((C,), jnp.int32),
            pltpu.VMEM((C, 2 * D), jnp.float32),
            pltpu.VMEM((C, 2 * D), jnp.float32),
            pltpu.VMEM((C, D), jnp.float32),
            pltpu.VMEM((C, DK), jnp.float32),
            pltpu.VMEM((C, ACC_W), jnp.float32),
            pltpu.SemaphoreType.DMA,
            pltpu.SemaphoreType.DMA,
            pltpu.SemaphoreType.DMA,
            pltpu.SemaphoreType.DMA,
        ],
    )
    def edge_kernel(qu_hbm, kv_hbm, g_hbm, ea_hbm, src_hbm, dst_hbm, zeros_hbm,
                    parts_hbm, acc, srcv, dstv, qu_v, kv_v, g_v, ea_v, msg_v,
                    sem0, sem1, sem2, sem3):
        cid = lax.axis_index("c")
        sid = lax.axis_index("s")
        wid = cid * NS + sid

        pltpu.sync_copy(zeros_hbm, acc.at[pl.ds(sid * NR, NR)])
        plsc.subcore_barrier()

        lane = lax.broadcasted_iota(jnp.int32, (16,), 0)

        def chunk_body(it, carry):
            base = wid * EP + it * C
            cp_s = pltpu.async_copy(src_hbm.at[pl.ds(base, C)], srcv, sem0)
            cp_d = pltpu.async_copy(dst_hbm.at[pl.ds(base, C)], dstv, sem1)
            cp_g = pltpu.async_copy(g_hbm.at[pl.ds(base, C)], g_v, sem2)
            cp_e = pltpu.async_copy(ea_hbm.at[pl.ds(base, C)], ea_v, sem3)
            cp_s.wait()
            cp_d.wait()
            gq = pltpu.async_copy(qu_hbm.at[dstv], qu_v, sem0)
            gk = pltpu.async_copy(kv_hbm.at[srcv], kv_v, sem1)
            gq.wait()
            gk.wait()
            cp_g.wait()
            cp_e.wait()

            def edge_body(i, c2):
                a = ea_v[i]
                evec = jnp.zeros((16,), jnp.float32)
                for h in range(HEADS):
                    o = h * DK
                    q = qu_v[i, pl.ds(o, DK)]
                    u = qu_v[i, pl.ds(D + o, DK)]
                    k = kv_v[i, pl.ds(o, DK)]
                    v = kv_v[i, pl.ds(D + o, DK)]
                    gm = g_v[i, pl.ds(o, DK)]
                    t = q * k
                    t = t + t * gm
                    t = t + a * u
                    lh = jnp.sum(t) * 0.25
                    eb = jnp.exp(jnp.broadcast_to(lh, (16,)))
                    msg_v[i, pl.ds(o, DK)] = eb * v
                    evec = jnp.where(lane == h, eb, evec)
                msg_v[i, pl.ds(D, 16)] = evec
                return c2

            lax.fori_loop(0, C, edge_body, 0)
            pltpu.sync_copy(msg_v, acc.at[dstv], add=True)
            return carry

        lax.fori_loop(0, CH, chunk_body, 0)
        plsc.subcore_barrier()
        pltpu.sync_copy(acc.at[pl.ds(sid * NR, NR)],
                        parts_hbm.at[cid, pl.ds(sid * NR, NR)])

    return edge_kernel


def kernel(x, edge_index, edge_attr, Wq, bq, Wk, bk, Wv, bv, Wf, bf, Wo, bo,
           ln_g, ln_b):
    B, N, D = x.shape
    E = edge_index.shape[1]
    x2 = x.reshape(N, D)
    src = edge_index[0]
    dst = edge_index[1]

    wfb_r = Wf[:, D:].reshape(DK, HEADS, DK)
    bd = jax.scipy.linalg.block_diag(
        *[wfb_r[:, h, :].T for h in range(HEADS)])
    wfg = Wf[:, :D]
    bfg = bf[:D].reshape(1, D)

    RN = 1000
    qu, kv = pl.pallas_call(
        _qkv_body,
        grid=(N // RN,),
        in_specs=[
            pl.BlockSpec((RN, D), lambda i: (i, 0)),
            pl.BlockSpec((D, D), lambda i: (0, 0)),
            pl.BlockSpec((D, D), lambda i: (0, 0)),
            pl.BlockSpec((D, D), lambda i: (0, 0)),
            pl.BlockSpec((D, D), lambda i: (0, 0)),
            pl.BlockSpec((1, D), lambda i: (0, 0)),
            pl.BlockSpec((1, D), lambda i: (0, 0)),
            pl.BlockSpec((1, D), lambda i: (0, 0)),
        ],
        out_specs=[
            pl.BlockSpec((RN, 2 * D), lambda i: (i, 0)),
            pl.BlockSpec((RN, 2 * D), lambda i: (i, 0)),
        ],
        out_shape=[
            jax.ShapeDtypeStruct((N, 2 * D), jnp.float32),
            jax.ShapeDtypeStruct((N, 2 * D), jnp.float32),
        ],
    )(x2, Wq, Wk, Wv, bd, bq.reshape(1, D), bk.reshape(1, D), bv.reshape(1, D))

    RE = 4000
    g = pl.pallas_call(
        _gamma_body,
        grid=(E // RE,),
        in_specs=[
            pl.BlockSpec((RE, DK), lambda i: (i, 0)),
            pl.BlockSpec((DK, D), lambda i: (0, 0)),
            pl.BlockSpec((1, D), lambda i: (0, 0)),
        ],
        out_specs=pl.BlockSpec((RE, D), lambda i: (i, 0)),
        out_shape=jax.ShapeDtypeStruct((E, D), jnp.float32),
    )(edge_attr, wfg, bfg)

    zeros_tile = jnp.zeros((N // 16, ACC_W), jnp.float32)
    parts = _make_edge_kernel(N, E, D)(qu, kv, g, edge_attr, src, dst,
                                       zeros_tile)

    out = pl.pallas_call(
        _combine_body,
        grid=(N // RN,),
        in_specs=[
            pl.BlockSpec((2, RN, ACC_W), lambda i: (0, i, 0)),
            pl.BlockSpec((RN, D), lambda i: (i, 0)),
            pl.BlockSpec((D, D), lambda i: (0, 0)),
            pl.BlockSpec((1, D), lambda i: (0, 0)),
            pl.BlockSpec((1, D), lambda i: (0, 0)),
            pl.BlockSpec((1, D), lambda i: (0, 0)),
        ],
        out_specs=pl.BlockSpec((RN, D), lambda i: (i, 0)),
        out_shape=jax.ShapeDtypeStruct((N, D), jnp.float32),
    )(parts, x2, Wo, bo.reshape(1, D), ln_g.reshape(1, D), ln_b.reshape(1, D))

    return out.reshape(B, N, D)


# speedup vs baseline: 37.6113x; 37.6113x over previous
import functools

import jax
import jax.numpy as jnp
from jax import lax
from jax.experimental import pallas as pl
from jax.experimental.pallas import tpu as pltpu
from jax.experimental.pallas import tpu_sc as plsc

HEADS = 8
DK = 16
HH = 64


def _qkv_body(x_ref, wq_ref, wk_ref, wv_ref, bd_ref, bq_ref, bk_ref, bv_ref,
              qu_ref, kv_ref):
    xb = x_ref[...]
    q = jnp.dot(xb, wq_ref[...], preferred_element_type=jnp.float32) + bq_ref[...]
    k = jnp.dot(xb, wk_ref[...], preferred_element_type=jnp.float32) + bk_ref[...]
    v = jnp.dot(xb, wv_ref[...], preferred_element_type=jnp.float32) + bv_ref[...]
    u = jnp.dot(q, bd_ref[...], preferred_element_type=jnp.float32)
    for c in range(2):
        qu_ref[c, :, 0:HH] = q[:, c * HH:(c + 1) * HH]
        qu_ref[c, :, HH:128] = u[:, c * HH:(c + 1) * HH]
        kv_ref[c, :, 0:HH] = k[:, c * HH:(c + 1) * HH]
        kv_ref[c, :, HH:128] = v[:, c * HH:(c + 1) * HH]


def _gamma_body(ea_ref, wfg_ref, bfg_ref, ga_ref):
    ea = ea_ref[...]
    gm = (jnp.dot(ea, wfg_ref[...], preferred_element_type=jnp.float32)
          + bfg_ref[...])
    z = jnp.zeros((ea.shape[0], 48), jnp.float32)
    for c in range(2):
        ga_ref[c, :, 0:HH] = gm[:, c * HH:(c + 1) * HH]
        ga_ref[c, :, HH:80] = ea
        ga_ref[c, :, 80:128] = z


def _combine_body(pm_ref, x_ref, woa_ref, wob_ref, bo_ref, lng_ref, lnb_ref,
                  o_ref):
    p0 = pm_ref[0]
    p1 = pm_ref[1]
    att0 = p0[:, 0:HH] / (p0[:, HH:128] + 1e-16)
    att1 = p1[:, 0:HH] / (p1[:, HH:128] + 1e-16)
    y = (jnp.dot(att0, woa_ref[...], preferred_element_type=jnp.float32)
         + jnp.dot(att1, wob_ref[...], preferred_element_type=jnp.float32)
         + bo_ref[...])
    res = x_ref[...] + y
    mean = jnp.mean(res, axis=1, keepdims=True)
    cen = res - mean
    var = jnp.mean(cen * cen, axis=1, keepdims=True)
    o_ref[...] = lng_ref[...] * cen * lax.rsqrt(var + 1e-5) + lnb_ref[...]


def _make_edge_kernel(N, E, D):
    info = plsc.get_sparse_core_info()
    NC, NS = info.num_cores, info.num_subcores
    EP = E // NS
    C = 80
    NP = -(-N // (8 * NS)) * (8 * NS)
    assert E % NS == 0 and EP % C == 0 and NC == 2
    CH = EP // C
    NR = NP // NS

    mesh = plsc.VectorSubcoreMesh(core_axis_name="c", subcore_axis_name="s")

    @functools.partial(
        pl.kernel,
        mesh=mesh,
        out_type=jax.ShapeDtypeStruct((NC, NP, 128), jnp.float32),
        scratch_types=[
            pltpu.VMEM_SHARED((NP, 128), jnp.float32),
            pltpu.VMEM((C,), jnp.int32),
            pltpu.VMEM((C,), jnp.int32),
            pltpu.VMEM((C,), jnp.int32),
            pltpu.VMEM((C, 128), jnp.float32),
            pltpu.VMEM((C, 128), jnp.float32),
            pltpu.VMEM((C, 128), jnp.float32),
            pltpu.VMEM((C, 128), jnp.float32),
            pltpu.SemaphoreType.DMA,
            pltpu.SemaphoreType.DMA,
            pltpu.SemaphoreType.DMA,
        ],
    )
    def edge_kernel(qu_hbm, kv_hbm, ga_hbm, src_hbm, dst_hbm, zm_hbm,
                    pm_hbm, accm, srcv, dstv, dstg, qu_v, kv_v, ga_v,
                    msg_v, sem0, sem1, sem2):
        cid = lax.axis_index("c")
        sid = lax.axis_index("s")

        row0 = pl.multiple_of(sid * NR, 8)
        pltpu.sync_copy(zm_hbm, accm.at[pl.ds(row0, NR)])
        plsc.subcore_barrier()

        lane = lax.broadcasted_iota(jnp.int32, (16,), 0)
        perm = [(lane + (1 << p)) & 15 for p in range(4)]
        off = cid * N

        def chunk_body(it, carry):
            base = sid * EP + it * C
            cp_s = pltpu.async_copy(src_hbm.at[pl.ds(base, C)], srcv, sem0)
            cp_d = pltpu.async_copy(dst_hbm.at[pl.ds(base, C)], dstv, sem1)
            cp_g = pltpu.async_copy(ga_hbm.at[cid, pl.ds(base, C)], ga_v, sem2)
            cp_s.wait()
            cp_d.wait()
            for g in range(C // 16):
                sl = pl.ds(g * 16, 16)
                srcv[sl] = srcv[sl] + off
                dstg[sl] = dstv[sl] + off
            gq = pltpu.async_copy(qu_hbm.at[dstg], qu_v, sem0)
            gk = pltpu.async_copy(kv_hbm.at[srcv], kv_v, sem1)
            gq.wait()
            gk.wait()
            cp_g.wait()

            def edge_body(i, c2):
                a = ga_v[i, pl.ds(HH, DK)]
                for h in range(4):
                    o = h * DK
                    q = qu_v[i, pl.ds(o, DK)]
                    u = qu_v[i, pl.ds(HH + o, DK)]
                    k = kv_v[i, pl.ds(o, DK)]
                    v = kv_v[i, pl.ds(HH + o, DK)]
                    gm = ga_v[i, pl.ds(o, DK)]
                    t = q * k
                    t = t + t * gm
                    t = t + a * u
                    for p in perm:
                        t = t + t.at[p].get(mode="promise_in_bounds")
                    eb = jnp.exp(t * 0.25)
                    msg_v[i, pl.ds(o, DK)] = eb * v
                    msg_v[i, pl.ds(HH + o, DK)] = eb
                return c2

            lax.fori_loop(0, C, edge_body, 0)
            pltpu.sync_copy(msg_v, accm.at[dstv], add=True)
            return carry

        lax.fori_loop(0, CH, chunk_body, 0)
        plsc.subcore_barrier()
        pltpu.sync_copy(accm.at[pl.ds(row0, NR)],
                        pm_hbm.at[cid, pl.ds(row0, NR)])

    return edge_kernel


def kernel(x, edge_index, edge_attr, Wq, bq, Wk, bk, Wv, bv, Wf, bf, Wo, bo,
           ln_g, ln_b):
    B, N, D = x.shape
    E = edge_index.shape[1]
    x2 = x.reshape(N, D)
    src = edge_index[0]
    dst = edge_index[1]

    wfb_r = Wf[:, D:].reshape(DK, HEADS, DK)
    bd = jax.scipy.linalg.block_diag(
        *[wfb_r[:, h, :].T for h in range(HEADS)])
    wfg = Wf[:, :D]
    bfg = bf[:D].reshape(1, D)
    woa = Wo[0:HH]
    wob = Wo[HH:D]

    RN = 1000
    qu, kv = pl.pallas_call(
        _qkv_body,
        grid=(N // RN,),
        in_specs=[
            pl.BlockSpec((RN, D), lambda i: (i, 0)),
            pl.BlockSpec((D, D), lambda i: (0, 0)),
            pl.BlockSpec((D, D), lambda i: (0, 0)),
            pl.BlockSpec((D, D), lambda i: (0, 0)),
            pl.BlockSpec((D, D), lambda i: (0, 0)),
            pl.BlockSpec((1, D), lambda i: (0, 0)),
            pl.BlockSpec((1, D), lambda i: (0, 0)),
            pl.BlockSpec((1, D), lambda i: (0, 0)),
        ],
        out_specs=[
            pl.BlockSpec((2, RN, 128), lambda i: (0, i, 0)),
            pl.BlockSpec((2, RN, 128), lambda i: (0, i, 0)),
        ],
        out_shape=[
            jax.ShapeDtypeStruct((2, N, 128), jnp.float32),
            jax.ShapeDtypeStruct((2, N, 128), jnp.float32),
        ],
    )(x2, Wq, Wk, Wv, bd, bq.reshape(1, D), bk.reshape(1, D), bv.reshape(1, D))
    qu = qu.reshape(2 * N, 128)
    kv = kv.reshape(2 * N, 128)

    RE = 4000
    ga = pl.pallas_call(
        _gamma_body,
        grid=(E // RE,),
        in_specs=[
            pl.BlockSpec((RE, DK), lambda i: (i, 0)),
            pl.BlockSpec((DK, D), lambda i: (0, 0)),
            pl.BlockSpec((1, D), lambda i: (0, 0)),
        ],
        out_specs=pl.BlockSpec((2, RE, 128), lambda i: (0, i, 0)),
        out_shape=jax.ShapeDtypeStruct((2, E, 128), jnp.float32),
    )(edge_attr, wfg, bfg)

    NP = -(-N // 128) * 128
    zeros_m = jnp.zeros((NP // 16, 128), jnp.float32)
    pm = _make_edge_kernel(N, E, D)(qu, kv, ga, src, dst, zeros_m)

    out = pl.pallas_call(
        _combine_body,
        grid=(N // RN,),
        in_specs=[
            pl.BlockSpec((2, RN, 128), lambda i: (0, i, 0)),
            pl.BlockSpec((RN, D), lambda i: (i, 0)),
            pl.BlockSpec((HH, D), lambda i: (0, 0)),
            pl.BlockSpec((HH, D), lambda i: (0, 0)),
            pl.BlockSpec((1, D), lambda i: (0, 0)),
            pl.BlockSpec((1, D), lambda i: (0, 0)),
            pl.BlockSpec((1, D), lambda i: (0, 0)),
        ],
        out_specs=pl.BlockSpec((RN, D), lambda i: (i, 0)),
        out_shape=jax.ShapeDtypeStruct((N, D), jnp.float32),
    )(pm, x2, woa, wob, bo.reshape(1, D), ln_g.reshape(1, D), ln_b.reshape(1, D))

    return out.reshape(B, N, D)
